# baseline (speedup 1.0000x reference)
import functools

import jax
import jax.numpy as jnp
from jax import lax
from jax.experimental import pallas as pl
from jax.experimental.pallas import tpu as pltpu
from jax.experimental.pallas import tpu_sc as plsc

_VOCAB = 1000000
_DIM = 32
_BATCH = 4096
_HIST = 200

_NC = 2
_NS = 16
_NW = _NC * _NS
_L = 16

_BB = _BATCH // _NW
_NRING = 4
_LOOK = 3
_NGROUPS = _HIST // _NRING


def _build():
    mesh = plsc.VectorSubcoreMesh(core_axis_name="c", subcore_axis_name="s")

    @functools.partial(
        pl.kernel,
        mesh=mesh,
        out_type=jax.ShapeDtypeStruct((_HIST, _DIM // 8, _NW, 8, _BB),
                                      jnp.float32),
        scratch_types=[
            pltpu.VMEM((_BB, _HIST), jnp.int32),
            pltpu.VMEM((_HIST, _BB), jnp.int32),
            pltpu.VMEM((_NRING, _BB, _DIM), jnp.float32),
            pltpu.VMEM((_NRING, _DIM // 8, 8, _BB), jnp.float32),
            pltpu.SemaphoreType.DMA((_NRING,)),
            pltpu.SemaphoreType.DMA((_NRING,)),
        ],
        compiler_params=pltpu.CompilerParams(use_tc_tiling_on_sc=False,
                                             needs_layout_passes=False),
    )
    def gather_kernel(idx_hbm, table_hbm, out_hbm, raw_v, idxt_v, rows_v,
                      outt_v, gsem, osem):
        wid = lax.axis_index("s") * _NC + lax.axis_index("c")
        base = wid * _BB
        pltpu.sync_copy(idx_hbm.at[pl.ds(base, _BB), :], raw_v)

        iota = lax.iota(jnp.int32, _L)
        row_idx = [iota + (k * _L) for k in range(_BB // _L)]

        def tbody(h, carry):
            col = jnp.zeros((_L,), jnp.int32) + h
            for k in range(_BB // _L):
                v = plsc.load_gather(raw_v, [row_idx[k], col])
                idxt_v[h, pl.ds(k * _L, _L)] = v
            return carry

        lax.fori_loop(0, _HIST, tbody, 0)

        def start_gather(h, b):
            pltpu.async_copy(table_hbm.at[idxt_v.at[h]], rows_v.at[b],
                             gsem.at[b])

        def wait_gather(h, b):
            pltpu.make_async_copy(table_hbm.at[idxt_v.at[h]], rows_v.at[b],
                                  gsem.at[b]).wait()

        def start_write(h, b):
            pltpu.async_copy(outt_v.at[b], out_hbm.at[h, :, wid], osem.at[b])

        def wait_write(h, b):
            pltpu.make_async_copy(outt_v.at[b], out_hbm.at[h, :, wid],
                                  osem.at[b]).wait()

        def transpose_rows(rb, ob):
            for td in range(_DIM // 8):
                for r in range(8):
                    d = jnp.full((_L,), td * 8 + r, jnp.int32)
                    for k in range(_BB // _L):
                        v = plsc.load_gather(rows_v.at[rb], [row_idx[k], d])
                        outt_v[ob, td, r, pl.ds(k * _L, _L)] = v

        for b in range(_LOOK):
            start_gather(b, b)

        def body(g, carry):
            for b in range(_NRING):
                h = g * _NRING + b
                wait_gather(h, b)

                @pl.when(g > 0)
                def _():
                    wait_write(h - _NRING, b)
                transpose_rows(b, b)
                start_write(h, b)

                hg = h + _LOOK
                bg = (b + _LOOK) % _NRING

                @pl.when(hg < _HIST)
                def _():
                    start_gather(hg, bg)
            return carry

        lax.fori_loop(0, _NGROUPS, body, 0)

        for b in range(_NRING):
            wait_write(_HIST - _NRING + b, b)

    return gather_kernel


_gather = _build()


def kernel(x, table):
    out5 = _gather(x.astype(jnp.int32), table)
    return out5.transpose(2, 4, 0, 1, 3).reshape(_BATCH, _HIST, _DIM)


# speedup vs baseline: 1.1607x; 1.1607x over previous
import functools

import jax
import jax.numpy as jnp
from jax import lax
from jax.experimental import pallas as pl
from jax.experimental.pallas import tpu as pltpu
from jax.experimental.pallas import tpu_sc as plsc

_VOCAB = 1000000
_DIM = 32
_BATCH = 4096
_HIST = 200

_NC = 2
_NS = 16
_NW = _NC * _NS
_L = 16

_BB = _BATCH // _NW
_NRING = 4
_LOOK = 3
_NGROUPS = _HIST // _NRING


def _build():
    mesh = plsc.VectorSubcoreMesh(core_axis_name="c", subcore_axis_name="s")

    @functools.partial(
        pl.kernel,
        mesh=mesh,
        out_type=jax.ShapeDtypeStruct((_HIST, _DIM // 8, _NW, 8, _BB),
                                      jnp.float32),
        scratch_types=[
            pltpu.VMEM((_BB, _HIST), jnp.int32),
            pltpu.VMEM((_HIST, _BB), jnp.int32),
            pltpu.VMEM((_NRING, _BB, _DIM), jnp.float32),
            pltpu.VMEM((_NRING, _DIM // 8, 8, _BB), jnp.float32),
            pltpu.SemaphoreType.DMA((_NRING,)),
            pltpu.SemaphoreType.DMA((_NRING,)),
        ],
        compiler_params=pltpu.CompilerParams(use_tc_tiling_on_sc=False,
                                             needs_layout_passes=False),
    )
    def gather_kernel(idx_hbm, table_hbm, out_hbm, raw_v, idxt_v, rows_v,
                      outt_v, gsem, osem):
        wid = lax.axis_index("s") * _NC + lax.axis_index("c")
        base = wid * _BB
        pltpu.sync_copy(idx_hbm.at[pl.ds(base, _BB), :], raw_v)

        iota = lax.iota(jnp.int32, _L)
        row_idx = [iota + (k * _L) for k in range(_BB // _L)]

        def tbody(h, carry):
            col = jnp.zeros((_L,), jnp.int32) + h
            for k in range(_BB // _L):
                v = plsc.load_gather(raw_v, [row_idx[k], col])
                idxt_v[h, pl.ds(k * _L, _L)] = v
            return carry

        lax.fori_loop(0, _HIST, tbody, 0)

        def start_gather(h, b):
            pltpu.async_copy(table_hbm.at[idxt_v.at[h]], rows_v.at[b],
                             gsem.at[b])

        def wait_gather(h, b):
            pltpu.make_async_copy(table_hbm.at[idxt_v.at[h]], rows_v.at[b],
                                  gsem.at[b]).wait()

        def start_write(h, b):
            pltpu.async_copy(outt_v.at[b], out_hbm.at[h, :, wid], osem.at[b])

        def wait_write(h, b):
            pltpu.make_async_copy(outt_v.at[b], out_hbm.at[h, :, wid],
                                  osem.at[b]).wait()

        def transpose_rows(rb, ob):
            for td in range(_DIM // 8):
                for rh in range(2):
                    batch = []
                    for r in range(rh * 4, rh * 4 + 4):
                        d = jnp.full((_L,), td * 8 + r, jnp.int32)
                        for k in range(_BB // _L):
                            v = plsc.load_gather(rows_v.at[rb],
                                                 [row_idx[k], d])
                            batch.append((r, k, v))
                    for r, k, v in batch:
                        outt_v[ob, td, r, pl.ds(k * _L, _L)] = v

        for b in range(_LOOK):
            start_gather(b, b)

        def body(g, carry):
            for b in range(_NRING):
                h = g * _NRING + b
                wait_gather(h, b)

                @pl.when(g > 0)
                def _():
                    wait_write(h - _NRING, b)
                transpose_rows(b, b)
                start_write(h, b)

                hg = h + _LOOK
                bg = (b + _LOOK) % _NRING

                @pl.when(hg < _HIST)
                def _():
                    start_gather(hg, bg)
            return carry

        lax.fori_loop(0, _NGROUPS, body, 0)

        for b in range(_NRING):
            wait_write(_HIST - _NRING + b, b)

    return gather_kernel


_gather = _build()


def kernel(x, table):
    out5 = _gather(x.astype(jnp.int32), table)
    return out5.transpose(2, 4, 0, 1, 3).reshape(_BATCH, _HIST, _DIM)


# speedup vs baseline: 1.6081x; 1.3855x over previous
import functools

import jax
import jax.numpy as jnp
from jax import lax
from jax.experimental import pallas as pl
from jax.experimental.pallas import tpu as pltpu
from jax.experimental.pallas import tpu_sc as plsc

_VOCAB = 1000000
_DIM = 32
_BATCH = 4096
_HIST = 200

_NC = 2
_NS = 16
_NW = _NC * _NS

_RPW = _BATCH // _NW
_NRING = 4
_LOOK = 3
_NGROUPS = _RPW // _NRING
_HT = _HIST // 8


def _build():
    mesh = plsc.VectorSubcoreMesh(core_axis_name="c", subcore_axis_name="s")

    @functools.partial(
        pl.kernel,
        mesh=mesh,
        out_type=jax.ShapeDtypeStruct((_BATCH, _HIST, 128), jnp.float32),
        scratch_types=[
            pltpu.VMEM((_RPW, _HIST), jnp.int32),
            pltpu.VMEM((_NRING, _HIST, _DIM), jnp.float32),
            pltpu.SemaphoreType.DMA((_NRING,)),
            pltpu.SemaphoreType.DMA((_NRING,)),
        ],
        compiler_params=pltpu.CompilerParams(use_tc_tiling_on_sc=False,
                                             needs_layout_passes=False),
    )
    def gather_kernel(idx_hbm, table_hbm, out_hbm, idx_v, rows_v, gsem, osem):
        wid = lax.axis_index("s") * _NC + lax.axis_index("c")
        base = wid * _RPW
        pltpu.sync_copy(idx_hbm.at[pl.ds(base, _RPW), :], idx_v)

        def start_gather(j, b):
            pltpu.async_copy(table_hbm.at[idx_v.at[j]], rows_v.at[b],
                             gsem.at[b])

        def wait_gather(j, b):
            pltpu.make_async_copy(table_hbm.at[idx_v.at[j]], rows_v.at[b],
                                  gsem.at[b]).wait()

        def start_write(j, b):
            pltpu.async_copy(rows_v.at[b],
                             out_hbm.at[base + j, :, pl.ds(0, _DIM)],
                             osem.at[b])

        def wait_write(j, b):
            pltpu.make_async_copy(rows_v.at[b],
                                  out_hbm.at[base + j, :, pl.ds(0, _DIM)],
                                  osem.at[b]).wait()

        for b in range(_LOOK):
            start_gather(b, b)

        def body(g, carry):
            for b in range(_NRING):
                j = g * _NRING + b
                wait_gather(j, b)
                start_write(j, b)
                jg = j + _LOOK
                bg = (b + _LOOK) % _NRING

                @pl.when(jg < _RPW)
                def _():
                    @pl.when(jg >= _NRING)
                    def _():
                        wait_write(jg - _NRING, bg)
                    start_gather(jg, bg)
            return carry

        lax.fori_loop(0, _NGROUPS, body, 0)

        for b in range(_NRING):
            j = _RPW - _NRING + b
            wait_write(j, b)

    return gather_kernel


_gather = _build()


def kernel(x, table):
    op = _gather(x.astype(jnp.int32), table)
    return op[:, :, :_DIM]


# speedup vs baseline: 1.6126x; 1.0028x over previous
import functools

import jax
import jax.numpy as jnp
from jax import lax
from jax.experimental import pallas as pl
from jax.experimental.pallas import tpu as pltpu
from jax.experimental.pallas import tpu_sc as plsc

_VOCAB = 1000000
_DIM = 32
_BATCH = 4096
_HIST = 200

_NC = 2
_NS = 16
_NW = _NC * _NS

_RPW = _BATCH // _NW
_NRING = 8
_LOOK = 6
_NGROUPS = _RPW // _NRING
_HT = _HIST // 8


def _build():
    mesh = plsc.VectorSubcoreMesh(core_axis_name="c", subcore_axis_name="s")

    @functools.partial(
        pl.kernel,
        mesh=mesh,
        out_type=jax.ShapeDtypeStruct((_BATCH, _HIST, 128), jnp.float32),
        scratch_types=[
            pltpu.VMEM((_RPW, _HIST), jnp.int32),
            pltpu.VMEM((_NRING, _HIST, _DIM), jnp.float32),
            pltpu.SemaphoreType.DMA((_NRING,)),
            pltpu.SemaphoreType.DMA((_NRING,)),
        ],
        compiler_params=pltpu.CompilerParams(use_tc_tiling_on_sc=False,
                                             needs_layout_passes=False),
    )
    def gather_kernel(idx_hbm, table_hbm, out_hbm, idx_v, rows_v, gsem, osem):
        wid = lax.axis_index("s") * _NC + lax.axis_index("c")
        base = wid * _RPW
        pltpu.sync_copy(idx_hbm.at[pl.ds(base, _RPW), :], idx_v)

        def start_gather(j, b):
            pltpu.async_copy(table_hbm.at[idx_v.at[j]], rows_v.at[b],
                             gsem.at[b])

        def wait_gather(j, b):
            pltpu.make_async_copy(table_hbm.at[idx_v.at[j]], rows_v.at[b],
                                  gsem.at[b]).wait()

        def start_write(j, b):
            pltpu.async_copy(rows_v.at[b],
                             out_hbm.at[base + j, :, pl.ds(0, _DIM)],
                             osem.at[b])

        def wait_write(j, b):
            pltpu.make_async_copy(rows_v.at[b],
                                  out_hbm.at[base + j, :, pl.ds(0, _DIM)],
                                  osem.at[b]).wait()

        for b in range(_LOOK):
            start_gather(b, b)

        def body(g, carry):
            for b in range(_NRING):
                j = g * _NRING + b
                wait_gather(j, b)
                start_write(j, b)
                jg = j + _LOOK
                bg = (b + _LOOK) % _NRING

                @pl.when(jg < _RPW)
                def _():
                    @pl.when(jg >= _NRING)
                    def _():
                        wait_write(jg - _NRING, bg)
                    start_gather(jg, bg)
            return carry

        lax.fori_loop(0, _NGROUPS, body, 0)

        for b in range(_NRING):
            j = _RPW - _NRING + b
            wait_write(j, b)

    return gather_kernel


_gather = _build()


def kernel(x, table):
    op = _gather(x.astype(jnp.int32), table)
    return op[:, :, :_DIM]
